# baseline (speedup 1.0000x reference)
import functools
import jax
import jax.numpy as jnp
from jax import lax
from jax.experimental import pallas as pl
from jax.experimental.pallas import tpu as pltpu
from jax.experimental.pallas import tpu_sc as plsc

H = 128
EPS_BN = 1e-5

_NW = 32
_CHUNK = 128


def _sc_gather(table, idx):
    v, h = table.shape
    b = idx.shape[0]
    bpw = b // _NW
    assert b % _NW == 0
    nfull = bpw // _CHUNK
    tail = bpw - nfull * _CHUNK
    npair = nfull // 2
    assert nfull % 2 == 0 and tail % 8 == 0
    mesh = plsc.VectorSubcoreMesh(core_axis_name="c", subcore_axis_name="s")

    @functools.partial(
        pl.kernel, mesh=mesh,
        out_type=jax.ShapeDtypeStruct((b, h), jnp.float32),
        scratch_types=[
            pltpu.VMEM((bpw,), jnp.int32),
            pltpu.VMEM((_CHUNK, h), jnp.float32),
            pltpu.VMEM((_CHUNK, h), jnp.float32),
            pltpu.SemaphoreType.DMA,
            pltpu.SemaphoreType.DMA,
        ],
    )
    def gather_k(tbl_hbm, idx_hbm, out_hbm, idx_v, buf0, buf1, sem0, sem1):
        wid = lax.axis_index("s") * 2 + lax.axis_index("c")
        base = wid * bpw
        pltpu.sync_copy(idx_hbm.at[pl.ds(base, bpw)], idx_v)
        pltpu.async_copy(tbl_hbm.at[idx_v.at[pl.ds(0, _CHUNK)]], buf0, sem0)

        def step2(c2, carry):
            off = c2 * (2 * _CHUNK)
            pltpu.async_copy(
                tbl_hbm.at[idx_v.at[pl.ds(off + _CHUNK, _CHUNK)]], buf1, sem1)
            pltpu.make_async_copy(
                tbl_hbm.at[idx_v.at[pl.ds(off, _CHUNK)]], buf0, sem0).wait()
            pltpu.sync_copy(buf0, out_hbm.at[pl.ds(base + off, _CHUNK)])

            @pl.when(c2 + 1 < npair)
            def _():
                pltpu.async_copy(
                    tbl_hbm.at[idx_v.at[pl.ds(off + 2 * _CHUNK, _CHUNK)]],
                    buf0, sem0)

            pltpu.make_async_copy(
                tbl_hbm.at[idx_v.at[pl.ds(off + _CHUNK, _CHUNK)]],
                buf1, sem1).wait()
            pltpu.sync_copy(
                buf1, out_hbm.at[pl.ds(base + off + _CHUNK, _CHUNK)])
            return carry

        lax.fori_loop(0, npair, step2, 0)
        if tail:
            toff = nfull * _CHUNK
            pltpu.async_copy(
                tbl_hbm.at[idx_v.at[pl.ds(toff, tail)]],
                buf0.at[pl.ds(0, tail)], sem0).wait()
            pltpu.sync_copy(
                buf0.at[pl.ds(0, tail)],
                out_hbm.at[pl.ds(base + toff, tail)])

    return gather_k(table, idx)


def _mm_stats_body(x_ref, w_ref, y_ref, s1_ref, s2_ref):
    i = pl.program_id(0)
    y = jnp.dot(x_ref[...], w_ref[...], preferred_element_type=jnp.float32)
    y_ref[...] = y

    @pl.when(i == 0)
    def _():
        s1_ref[...] = jnp.zeros_like(s1_ref)
        s2_ref[...] = jnp.zeros_like(s2_ref)

    s1_ref[...] += jnp.sum(y, axis=0, keepdims=True)
    s2_ref[...] += jnp.sum(y * y, axis=0, keepdims=True)


def _mm_stats(x, w, block):
    r, k = x.shape
    return pl.pallas_call(
        _mm_stats_body,
        grid=(r // block,),
        in_specs=[
            pl.BlockSpec((block, k), lambda i: (i, 0)),
            pl.BlockSpec((k, H), lambda i: (0, 0)),
        ],
        out_specs=[
            pl.BlockSpec((block, H), lambda i: (i, 0)),
            pl.BlockSpec((1, H), lambda i: (0, 0)),
            pl.BlockSpec((1, H), lambda i: (0, 0)),
        ],
        out_shape=[
            jax.ShapeDtypeStruct((r, H), jnp.float32),
            jax.ShapeDtypeStruct((1, H), jnp.float32),
            jax.ShapeDtypeStruct((1, H), jnp.float32),
        ],
    )(x, w)


def _bnrelu_mm_stats_body(y_ref, sc_ref, sh_ref, w_ref, z_ref, s1_ref, s2_ref):
    i = pl.program_id(0)
    h = jax.nn.relu(y_ref[...] * sc_ref[...] + sh_ref[...])
    z = jnp.dot(h, w_ref[...], preferred_element_type=jnp.float32)
    z_ref[...] = z

    @pl.when(i == 0)
    def _():
        s1_ref[...] = jnp.zeros_like(s1_ref)
        s2_ref[...] = jnp.zeros_like(s2_ref)

    s1_ref[...] += jnp.sum(z, axis=0, keepdims=True)
    s2_ref[...] += jnp.sum(z * z, axis=0, keepdims=True)


def _bnrelu_mm_stats(y, sc, sh, w, block):
    r, k = y.shape
    return pl.pallas_call(
        _bnrelu_mm_stats_body,
        grid=(r // block,),
        in_specs=[
            pl.BlockSpec((block, k), lambda i: (i, 0)),
            pl.BlockSpec((1, k), lambda i: (0, 0)),
            pl.BlockSpec((1, k), lambda i: (0, 0)),
            pl.BlockSpec((k, H), lambda i: (0, 0)),
        ],
        out_specs=[
            pl.BlockSpec((block, H), lambda i: (i, 0)),
            pl.BlockSpec((1, H), lambda i: (0, 0)),
            pl.BlockSpec((1, H), lambda i: (0, 0)),
        ],
        out_shape=[
            jax.ShapeDtypeStruct((r, H), jnp.float32),
            jax.ShapeDtypeStruct((1, H), jnp.float32),
            jax.ShapeDtypeStruct((1, H), jnp.float32),
        ],
    )(y, sc, sh, w)


def _bnrelu_body(y_ref, sc_ref, sh_ref, o_ref):
    o_ref[...] = jax.nn.relu(y_ref[...] * sc_ref[...] + sh_ref[...])


def _bnrelu(y, sc, sh, block):
    r, k = y.shape
    return pl.pallas_call(
        _bnrelu_body,
        grid=(r // block,),
        in_specs=[
            pl.BlockSpec((block, k), lambda i: (i, 0)),
            pl.BlockSpec((1, k), lambda i: (0, 0)),
            pl.BlockSpec((1, k), lambda i: (0, 0)),
        ],
        out_specs=pl.BlockSpec((block, k), lambda i: (i, 0)),
        out_shape=jax.ShapeDtypeStruct((r, k), jnp.float32),
    )(y, sc, sh)


def _stats_body(z_ref, s1_ref, s2_ref):
    i = pl.program_id(0)
    z = z_ref[...]

    @pl.when(i == 0)
    def _():
        s1_ref[...] = jnp.zeros_like(s1_ref)
        s2_ref[...] = jnp.zeros_like(s2_ref)

    s1_ref[...] += jnp.sum(z, axis=0, keepdims=True)
    s2_ref[...] += jnp.sum(z * z, axis=0, keepdims=True)


def _stats(z, block):
    r, k = z.shape
    return pl.pallas_call(
        _stats_body,
        grid=(r // block,),
        in_specs=[pl.BlockSpec((block, k), lambda i: (i, 0))],
        out_specs=[
            pl.BlockSpec((1, k), lambda i: (0, 0)),
            pl.BlockSpec((1, k), lambda i: (0, 0)),
        ],
        out_shape=[
            jax.ShapeDtypeStruct((1, k), jnp.float32),
            jax.ShapeDtypeStruct((1, k), jnp.float32),
        ],
    )(z)


def _bn_coeffs(s1, s2, n_rows, g, b):
    mu = s1[0] / n_rows
    var = s2[0] / n_rows - mu * mu
    rstd = jax.lax.rsqrt(var + EPS_BN)
    sc = g * rstd
    sh = b - mu * sc
    return sc[None, :], sh[None, :]


def kernel(node_rep, edge_rep, node2edge_index,
           W_lvl1, g_lvl1, b_lvl1,
           W_lvl2a, g_lvl2a, b_lvl2a,
           W_lvl2b, g_lvl2b, b_lvl2b,
           W_lifta, g_lifta, b_lifta,
           W_liftb, g_liftb, b_liftb,
           eps1, eps2):
    N = node_rep.shape[0]
    E = edge_rep.shape[0]
    i0 = node2edge_index[0]
    i1 = node2edge_index[1]
    M = i0.shape[0]

    W1a = W_lvl1[:, :H].T
    W1b = W_lvl1[:, H:].T
    P, _, _ = _mm_stats(node_rep, W1a, 1000)
    Q, _, _ = _mm_stats(edge_rep, W1b, 2000)

    i0 = i0.astype(jnp.int32)
    i1 = i1.astype(jnp.int32)
    R = jnp.concatenate([P, node_rep], axis=1)
    Rg = _sc_gather(R, i0)
    z = Rg[:, :H] + _sc_gather(Q, i1)
    s1, s2 = _stats(z, 4000)
    sc1, sh1 = _bn_coeffs(s1, s2, M, g_lvl1, b_lvl1)
    msg = _bnrelu(z, sc1, sh1, 4000)

    node_val = Rg[:, H:]
    S = jax.ops.segment_sum(msg, i1, num_segments=E)
    lift_aggr = jax.ops.segment_sum(node_val, i1, num_segments=E)
    D = _sc_gather(S, i1) - msg
    lvl_aggr = jax.ops.segment_sum(D, i0, num_segments=N)

    x = (1.0 + eps1) * node_rep + lvl_aggr
    a1, t1, t2 = _mm_stats(x, W_lvl2a.T, 1000)
    sca, sha = _bn_coeffs(t1, t2, N, g_lvl2a, b_lvl2a)
    a2, u1, u2 = _bnrelu_mm_stats(a1, sca, sha, W_lvl2b.T, 1000)
    scb, shb = _bn_coeffs(u1, u2, N, g_lvl2b, b_lvl2b)
    node_out = _bnrelu(a2, scb, shb, 1000)

    y = (1.0 + eps2) * edge_rep + lift_aggr
    e1, v1, v2 = _mm_stats(y, W_lifta.T, 2000)
    sce, she = _bn_coeffs(v1, v2, E, g_lifta, b_lifta)
    e2, w1, w2 = _bnrelu_mm_stats(e1, sce, she, W_liftb.T, 2000)
    scf, shf = _bn_coeffs(w1, w2, E, g_liftb, b_liftb)
    edge_out = _bnrelu(e2, scf, shf, 2000)

    return (node_out, edge_out)


# speedup vs baseline: 1.0051x; 1.0051x over previous
import functools
import jax
import jax.numpy as jnp
from jax import lax
from jax.experimental import pallas as pl
from jax.experimental.pallas import tpu as pltpu
from jax.experimental.pallas import tpu_sc as plsc

H = 128
EPS_BN = 1e-5

_NW = 32
_CHUNK = 128


def _sc_gather(table, idx):
    v, h = table.shape
    b = idx.shape[0]
    bpw = b // _NW
    assert b % _NW == 0
    nfull = bpw // _CHUNK
    tail = bpw - nfull * _CHUNK
    npair = nfull // 2
    assert nfull % 2 == 0 and tail % 8 == 0
    mesh = plsc.VectorSubcoreMesh(core_axis_name="c", subcore_axis_name="s")

    @functools.partial(
        pl.kernel, mesh=mesh,
        out_type=jax.ShapeDtypeStruct((b, h), jnp.float32),
        scratch_types=[
            pltpu.VMEM((bpw,), jnp.int32),
            pltpu.VMEM((_CHUNK, h), jnp.float32),
            pltpu.VMEM((_CHUNK, h), jnp.float32),
            pltpu.SemaphoreType.DMA,
            pltpu.SemaphoreType.DMA,
        ],
    )
    def gather_k(tbl_hbm, idx_hbm, out_hbm, idx_v, buf0, buf1, sem0, sem1):
        wid = lax.axis_index("s") * 2 + lax.axis_index("c")
        base = wid * bpw
        pltpu.sync_copy(idx_hbm.at[pl.ds(base, bpw)], idx_v)
        pltpu.async_copy(tbl_hbm.at[idx_v.at[pl.ds(0, _CHUNK)]], buf0, sem0)

        def step2(c2, carry):
            off = c2 * (2 * _CHUNK)
            pltpu.async_copy(
                tbl_hbm.at[idx_v.at[pl.ds(off + _CHUNK, _CHUNK)]], buf1, sem1)
            pltpu.make_async_copy(
                tbl_hbm.at[idx_v.at[pl.ds(off, _CHUNK)]], buf0, sem0).wait()
            pltpu.sync_copy(buf0, out_hbm.at[pl.ds(base + off, _CHUNK)])

            @pl.when(c2 + 1 < npair)
            def _():
                pltpu.async_copy(
                    tbl_hbm.at[idx_v.at[pl.ds(off + 2 * _CHUNK, _CHUNK)]],
                    buf0, sem0)

            pltpu.make_async_copy(
                tbl_hbm.at[idx_v.at[pl.ds(off + _CHUNK, _CHUNK)]],
                buf1, sem1).wait()
            pltpu.sync_copy(
                buf1, out_hbm.at[pl.ds(base + off + _CHUNK, _CHUNK)])
            return carry

        lax.fori_loop(0, npair, step2, 0)
        if tail:
            toff = nfull * _CHUNK
            pltpu.async_copy(
                tbl_hbm.at[idx_v.at[pl.ds(toff, tail)]],
                buf0.at[pl.ds(0, tail)], sem0).wait()
            pltpu.sync_copy(
                buf0.at[pl.ds(0, tail)],
                out_hbm.at[pl.ds(base + toff, tail)])

    return gather_k(table, idx)


def _mm_stats_body(x_ref, w_ref, y_ref, s1_ref, s2_ref):
    i = pl.program_id(0)
    y = jnp.dot(x_ref[...], w_ref[...], preferred_element_type=jnp.float32)
    y_ref[...] = y

    @pl.when(i == 0)
    def _():
        s1_ref[...] = jnp.zeros_like(s1_ref)
        s2_ref[...] = jnp.zeros_like(s2_ref)

    s1_ref[...] += jnp.sum(y, axis=0, keepdims=True)
    s2_ref[...] += jnp.sum(y * y, axis=0, keepdims=True)


def _mm_stats(x, w, block):
    r, k = x.shape
    return pl.pallas_call(
        _mm_stats_body,
        grid=(r // block,),
        in_specs=[
            pl.BlockSpec((block, k), lambda i: (i, 0)),
            pl.BlockSpec((k, H), lambda i: (0, 0)),
        ],
        out_specs=[
            pl.BlockSpec((block, H), lambda i: (i, 0)),
            pl.BlockSpec((1, H), lambda i: (0, 0)),
            pl.BlockSpec((1, H), lambda i: (0, 0)),
        ],
        out_shape=[
            jax.ShapeDtypeStruct((r, H), jnp.float32),
            jax.ShapeDtypeStruct((1, H), jnp.float32),
            jax.ShapeDtypeStruct((1, H), jnp.float32),
        ],
    )(x, w)


def _bnrelu_mm_stats_body(y_ref, sc_ref, sh_ref, w_ref, z_ref, s1_ref, s2_ref):
    i = pl.program_id(0)
    h = jax.nn.relu(y_ref[...] * sc_ref[...] + sh_ref[...])
    z = jnp.dot(h, w_ref[...], preferred_element_type=jnp.float32)
    z_ref[...] = z

    @pl.when(i == 0)
    def _():
        s1_ref[...] = jnp.zeros_like(s1_ref)
        s2_ref[...] = jnp.zeros_like(s2_ref)

    s1_ref[...] += jnp.sum(z, axis=0, keepdims=True)
    s2_ref[...] += jnp.sum(z * z, axis=0, keepdims=True)


def _bnrelu_mm_stats(y, sc, sh, w, block):
    r, k = y.shape
    return pl.pallas_call(
        _bnrelu_mm_stats_body,
        grid=(r // block,),
        in_specs=[
            pl.BlockSpec((block, k), lambda i: (i, 0)),
            pl.BlockSpec((1, k), lambda i: (0, 0)),
            pl.BlockSpec((1, k), lambda i: (0, 0)),
            pl.BlockSpec((k, H), lambda i: (0, 0)),
        ],
        out_specs=[
            pl.BlockSpec((block, H), lambda i: (i, 0)),
            pl.BlockSpec((1, H), lambda i: (0, 0)),
            pl.BlockSpec((1, H), lambda i: (0, 0)),
        ],
        out_shape=[
            jax.ShapeDtypeStruct((r, H), jnp.float32),
            jax.ShapeDtypeStruct((1, H), jnp.float32),
            jax.ShapeDtypeStruct((1, H), jnp.float32),
        ],
    )(y, sc, sh, w)


def _bnrelu_body(y_ref, sc_ref, sh_ref, o_ref):
    o_ref[...] = jax.nn.relu(y_ref[...] * sc_ref[...] + sh_ref[...])


def _bnrelu(y, sc, sh, block):
    r, k = y.shape
    return pl.pallas_call(
        _bnrelu_body,
        grid=(r // block,),
        in_specs=[
            pl.BlockSpec((block, k), lambda i: (i, 0)),
            pl.BlockSpec((1, k), lambda i: (0, 0)),
            pl.BlockSpec((1, k), lambda i: (0, 0)),
        ],
        out_specs=pl.BlockSpec((block, k), lambda i: (i, 0)),
        out_shape=jax.ShapeDtypeStruct((r, k), jnp.float32),
    )(y, sc, sh)


def _stats_body(z_ref, s1_ref, s2_ref):
    i = pl.program_id(0)
    z = z_ref[...]

    @pl.when(i == 0)
    def _():
        s1_ref[...] = jnp.zeros_like(s1_ref)
        s2_ref[...] = jnp.zeros_like(s2_ref)

    s1_ref[...] += jnp.sum(z, axis=0, keepdims=True)
    s2_ref[...] += jnp.sum(z * z, axis=0, keepdims=True)


def _stats(z, block):
    r, k = z.shape
    return pl.pallas_call(
        _stats_body,
        grid=(r // block,),
        in_specs=[pl.BlockSpec((block, k), lambda i: (i, 0))],
        out_specs=[
            pl.BlockSpec((1, k), lambda i: (0, 0)),
            pl.BlockSpec((1, k), lambda i: (0, 0)),
        ],
        out_shape=[
            jax.ShapeDtypeStruct((1, k), jnp.float32),
            jax.ShapeDtypeStruct((1, k), jnp.float32),
        ],
    )(z)


def _bn_coeffs(s1, s2, n_rows, g, b):
    mu = s1[0] / n_rows
    var = s2[0] / n_rows - mu * mu
    rstd = jax.lax.rsqrt(var + EPS_BN)
    sc = g * rstd
    sh = b - mu * sc
    return sc[None, :], sh[None, :]


def kernel(node_rep, edge_rep, node2edge_index,
           W_lvl1, g_lvl1, b_lvl1,
           W_lvl2a, g_lvl2a, b_lvl2a,
           W_lvl2b, g_lvl2b, b_lvl2b,
           W_lifta, g_lifta, b_lifta,
           W_liftb, g_liftb, b_liftb,
           eps1, eps2):
    N = node_rep.shape[0]
    E = edge_rep.shape[0]
    i0 = node2edge_index[0]
    i1 = node2edge_index[1]
    M = i0.shape[0]

    W1a = W_lvl1[:, :H].T
    W1b = W_lvl1[:, H:].T
    P, _, _ = _mm_stats(node_rep, W1a, 1000)
    Q, _, _ = _mm_stats(edge_rep, W1b, 2000)

    i0 = i0.astype(jnp.int32)
    i1 = i1.astype(jnp.int32)
    z = _sc_gather(P, i0) + _sc_gather(Q, i1)
    s1, s2 = _stats(z, 4000)
    sc1, sh1 = _bn_coeffs(s1, s2, M, g_lvl1, b_lvl1)
    msg = _bnrelu(z, sc1, sh1, 4000)

    node_val = _sc_gather(node_rep, i0)
    S = jax.ops.segment_sum(msg, i1, num_segments=E)
    lift_aggr = jax.ops.segment_sum(node_val, i1, num_segments=E)
    D = _sc_gather(S, i1) - msg
    lvl_aggr = jax.ops.segment_sum(D, i0, num_segments=N)

    x = (1.0 + eps1) * node_rep + lvl_aggr
    a1, t1, t2 = _mm_stats(x, W_lvl2a.T, 1000)
    sca, sha = _bn_coeffs(t1, t2, N, g_lvl2a, b_lvl2a)
    a2, u1, u2 = _bnrelu_mm_stats(a1, sca, sha, W_lvl2b.T, 1000)
    scb, shb = _bn_coeffs(u1, u2, N, g_lvl2b, b_lvl2b)
    node_out = _bnrelu(a2, scb, shb, 1000)

    y = (1.0 + eps2) * edge_rep + lift_aggr
    e1, v1, v2 = _mm_stats(y, W_lifta.T, 2000)
    sce, she = _bn_coeffs(v1, v2, E, g_lifta, b_lifta)
    e2, w1, w2 = _bnrelu_mm_stats(e1, sce, she, W_liftb.T, 2000)
    scf, shf = _bn_coeffs(w1, w2, E, g_liftb, b_liftb)
    edge_out = _bnrelu(e2, scf, shf, 2000)

    return (node_out, edge_out)
